# Initial kernel scaffold; baseline (speedup 1.0000x reference)
import jax
import jax.numpy as jnp
from jax.experimental import pallas as pl


def kernel(x, e_i_ts):
    raise NotImplementedError("write your pallas kernel here")


# speedup vs baseline: 1.6604x; 1.6604x over previous
import functools

import jax
import jax.numpy as jnp
from jax import lax
from jax.experimental import pallas as pl
from jax.experimental.pallas import tpu as pltpu
from jax.experimental.pallas import tpu_sc as plsc

TOK = 256
NUM_CODES = 8192
DIM = 64


def _vq_argmin_body(x_ref, e_ref, idx_ref, e2_ref):
    @pl.when(pl.program_id(0) == 0)
    def _():
        e = e_ref[...]
        e2_ref[...] = jnp.sum(e * e, axis=0, keepdims=True)

    xb = x_ref[0]
    xe = lax.dot_general(
        xb, e_ref[...], (((0,), (0,)), ((), ())),
        preferred_element_type=jnp.float32,
    )
    s = e2_ref[...] - 2.0 * xe
    idx_ref[0, 0, :] = jnp.argmin(s, axis=1).astype(jnp.int32)


GATHER_D = 128


def _gather_sc(table, idx):
    n = idx.shape[0]
    info = plsc.get_sparse_core_info()
    nw = info.num_cores * info.num_subcores
    b_per_w = n // nw
    mesh = plsc.VectorSubcoreMesh(core_axis_name="c", subcore_axis_name="s")

    @functools.partial(
        pl.kernel,
        mesh=mesh,
        out_type=jax.ShapeDtypeStruct((n, GATHER_D), jnp.float32),
        scratch_types=[
            pltpu.VMEM((b_per_w,), jnp.int32),
            pltpu.VMEM((b_per_w, GATHER_D), jnp.float32),
            pltpu.SemaphoreType.DMA,
        ],
    )
    def k(table_hbm, idx_hbm, out_hbm, idx_v, rows_v, sem):
        wid = lax.axis_index("s") * info.num_cores + lax.axis_index("c")
        base = wid * b_per_w
        pltpu.sync_copy(idx_hbm.at[pl.ds(base, b_per_w)], idx_v)
        pltpu.async_copy(table_hbm.at[idx_v], rows_v, sem).wait()
        pltpu.sync_copy(rows_v, out_hbm.at[pl.ds(base, b_per_w)])

    return k(table, idx)


def kernel(x, e_i_ts):
    B, C, H, W = x.shape
    x3 = x.reshape(B, C, H * W)
    n = B * H * W
    grid = n // TOK
    pb = (H * W) // TOK
    idx = pl.pallas_call(
        _vq_argmin_body,
        grid=(grid,),
        in_specs=[
            pl.BlockSpec((1, C, TOK), lambda i: (i // pb, 0, i % pb)),
            pl.BlockSpec((C, NUM_CODES), lambda i: (0, 0)),
        ],
        out_specs=pl.BlockSpec((1, 1, TOK), lambda i: (i, 0, 0)),
        out_shape=jax.ShapeDtypeStruct((grid, 1, TOK), jnp.int32),
        scratch_shapes=[pltpu.VMEM((1, NUM_CODES), jnp.float32)],
    )(x3, e_i_ts)
    flat_idx = idx.reshape(n)
    table = jnp.pad(e_i_ts.T, ((0, 0), (0, GATHER_D - C)))
    quant = _gather_sc(table, flat_idx)
    out = quant.reshape(B, H, W, GATHER_D)[..., :C].transpose(0, 3, 1, 2)
    return out


# speedup vs baseline: 1.9949x; 1.2015x over previous
import functools

import jax
import jax.numpy as jnp
from jax import lax
from jax.experimental import pallas as pl
from jax.experimental.pallas import tpu as pltpu
from jax.experimental.pallas import tpu_sc as plsc

TOK = 512
NUM_CODES = 8192
DIM = 64


def _vq_argmin_body(x_ref, e_ref, idx_ref, e2_ref):
    @pl.when(pl.program_id(0) == 0)
    def _():
        e = e_ref[...]
        e2_ref[...] = jnp.sum(e * e, axis=0, keepdims=True)

    xb = x_ref[0] * -2.0
    xe = lax.dot_general(
        xb, e_ref[...], (((0,), (0,)), ((), ())),
        preferred_element_type=jnp.float32,
    )
    s = e2_ref[...] + xe
    idx_ref[0, 0, :] = jnp.argmin(s, axis=1).astype(jnp.int32)


GATHER_D = 128


def _gather_sc(table, idx):
    n = idx.shape[0]
    info = plsc.get_sparse_core_info()
    nw = info.num_cores * info.num_subcores
    b_per_w = n // nw
    mesh = plsc.VectorSubcoreMesh(core_axis_name="c", subcore_axis_name="s")

    @functools.partial(
        pl.kernel,
        mesh=mesh,
        out_type=jax.ShapeDtypeStruct((n, GATHER_D), jnp.float32),
        scratch_types=[
            pltpu.VMEM((b_per_w,), jnp.int32),
            pltpu.VMEM((b_per_w, GATHER_D), jnp.float32),
            pltpu.SemaphoreType.DMA,
        ],
    )
    def k(table_hbm, idx_hbm, out_hbm, idx_v, rows_v, sem):
        wid = lax.axis_index("s") * info.num_cores + lax.axis_index("c")
        base = wid * b_per_w
        pltpu.sync_copy(idx_hbm.at[pl.ds(base, b_per_w)], idx_v)
        pltpu.async_copy(table_hbm.at[idx_v], rows_v, sem).wait()
        pltpu.sync_copy(rows_v, out_hbm.at[pl.ds(base, b_per_w)])

    return k(table, idx)


def kernel(x, e_i_ts):
    B, C, H, W = x.shape
    x3 = x.reshape(B, C, H * W)
    n = B * H * W
    grid = n // TOK
    pb = (H * W) // TOK
    idx = pl.pallas_call(
        _vq_argmin_body,
        grid=(grid,),
        in_specs=[
            pl.BlockSpec((1, C, TOK), lambda i: (i // pb, 0, i % pb)),
            pl.BlockSpec((C, NUM_CODES), lambda i: (0, 0)),
        ],
        out_specs=pl.BlockSpec((1, 1, TOK), lambda i: (i, 0, 0)),
        out_shape=jax.ShapeDtypeStruct((grid, 1, TOK), jnp.int32),
        scratch_shapes=[pltpu.VMEM((1, NUM_CODES), jnp.float32)],
    )(x3, e_i_ts)
    flat_idx = idx.reshape(n)
    table = jnp.pad(e_i_ts.T, ((0, 0), (0, GATHER_D - C)))
    quant = _gather_sc(table, flat_idx)
    out = quant.reshape(B, H, W, GATHER_D)[..., :C].transpose(0, 3, 1, 2)
    return out


# speedup vs baseline: 2.0062x; 1.0057x over previous
import functools

import jax
import jax.numpy as jnp
from jax import lax
from jax.experimental import pallas as pl
from jax.experimental.pallas import tpu as pltpu
from jax.experimental.pallas import tpu_sc as plsc

TOK = 1024
NUM_CODES = 8192
DIM = 64


def _vq_argmin_body(x_ref, e_ref, idx_ref, e2_ref):
    @pl.when(pl.program_id(0) == 0)
    def _():
        e = e_ref[...]
        e2_ref[...] = jnp.sum(e * e, axis=0, keepdims=True)

    xb = x_ref[0] * -2.0
    xe = lax.dot_general(
        xb, e_ref[...], (((0,), (0,)), ((), ())),
        preferred_element_type=jnp.float32,
    )
    s = e2_ref[...] + xe
    idx_ref[0, 0, :] = jnp.argmin(s, axis=1).astype(jnp.int32)


GATHER_D = 128


def _gather_sc(table, idx):
    n = idx.shape[0]
    info = plsc.get_sparse_core_info()
    nw = info.num_cores * info.num_subcores
    b_per_w = n // nw
    mesh = plsc.VectorSubcoreMesh(core_axis_name="c", subcore_axis_name="s")

    @functools.partial(
        pl.kernel,
        mesh=mesh,
        out_type=jax.ShapeDtypeStruct((n, GATHER_D), jnp.float32),
        scratch_types=[
            pltpu.VMEM((b_per_w,), jnp.int32),
            pltpu.VMEM((b_per_w, GATHER_D), jnp.float32),
            pltpu.SemaphoreType.DMA,
        ],
    )
    def k(table_hbm, idx_hbm, out_hbm, idx_v, rows_v, sem):
        wid = lax.axis_index("s") * info.num_cores + lax.axis_index("c")
        base = wid * b_per_w
        pltpu.sync_copy(idx_hbm.at[pl.ds(base, b_per_w)], idx_v)
        pltpu.async_copy(table_hbm.at[idx_v], rows_v, sem).wait()
        pltpu.sync_copy(rows_v, out_hbm.at[pl.ds(base, b_per_w)])

    return k(table, idx)


def kernel(x, e_i_ts):
    B, C, H, W = x.shape
    x3 = x.reshape(B, C, H * W)
    n = B * H * W
    grid = n // TOK
    pb = (H * W) // TOK
    idx = pl.pallas_call(
        _vq_argmin_body,
        grid=(grid,),
        in_specs=[
            pl.BlockSpec((1, C, TOK), lambda i: (i // pb, 0, i % pb)),
            pl.BlockSpec((C, NUM_CODES), lambda i: (0, 0)),
        ],
        out_specs=pl.BlockSpec((1, 1, TOK), lambda i: (i, 0, 0)),
        out_shape=jax.ShapeDtypeStruct((grid, 1, TOK), jnp.int32),
        scratch_shapes=[pltpu.VMEM((1, NUM_CODES), jnp.float32)],
    )(x3, e_i_ts)
    flat_idx = idx.reshape(n)
    table = jnp.pad(e_i_ts.T, ((0, 0), (0, GATHER_D - C)))
    quant = _gather_sc(table, flat_idx)
    out = quant.reshape(B, H, W, GATHER_D)[..., :C].transpose(0, 3, 1, 2)
    return out
